# baseline (speedup 1.0000x reference)
import functools

import jax
import jax.numpy as jnp
from jax import lax
from jax.experimental import pallas as pl
from jax.experimental.pallas import tpu as pltpu
from jax.experimental.pallas import tpu_sc as plsc

VOCAB = 1000000
D_MODEL = 64
PAD_IDX = 0
GRP = 128
GRP_PER_CHUNK = 2
BTC = 5120


def _sc_gather_packed(table2, idx3d, n_rows, nw):
  g_per_w = idx3d.shape[1]
  chunks = g_per_w // GRP_PER_CHUNK
  rows_chunk = GRP * GRP_PER_CHUNK
  mesh = plsc.VectorSubcoreMesh(core_axis_name="c", subcore_axis_name="s")

  @functools.partial(
      pl.kernel,
      out_type=jax.ShapeDtypeStruct((n_rows, 2 * D_MODEL), jnp.float32),
      mesh=mesh,
      scratch_types=[
          pltpu.VMEM((g_per_w, GRP), jnp.int32),
          pltpu.VMEM((rows_chunk, 2 * D_MODEL), jnp.float32),
          pltpu.VMEM((rows_chunk, 2 * D_MODEL), jnp.float32),
          pltpu.SemaphoreType.DMA,
          pltpu.SemaphoreType.DMA,
      ],
  )
  def k(table_hbm, idx_hbm, out_hbm, idx_v, rows_a, rows_b, sem_a, sem_b):
    wid = lax.axis_index("s") * plsc.get_sparse_core_info().num_cores + \
        lax.axis_index("c")
    pltpu.sync_copy(idx_hbm.at[wid], idx_v)
    row_base = wid * g_per_w * GRP

    def gather_chunk(c, rows_v, sem):
      copies = []
      for j in range(GRP_PER_CHUNK):
        copies.append(pltpu.async_copy(
            table_hbm.at[idx_v.at[c * GRP_PER_CHUNK + j]],
            rows_v.at[pl.ds(j * GRP, GRP)], sem))
      return copies

    def drain_and_store(c, copies, rows_v):
      for cp in copies:
        cp.wait()
      p0 = pl.multiple_of(row_base + c * rows_chunk, 8)
      pltpu.sync_copy(rows_v, out_hbm.at[pl.ds(p0, rows_chunk)])

    copies = gather_chunk(0, rows_a, sem_a)
    for c in range(chunks):
      buf = rows_a if c % 2 == 0 else rows_b
      nbuf, nsem = (rows_b, sem_b) if c % 2 == 0 else (rows_a, sem_a)
      ncopies = gather_chunk(c + 1, nbuf, nsem) if c + 1 < chunks else []
      drain_and_store(c, copies, buf)
      copies = ncopies

  return k(table2, idx3d)


def _tc_finish_body(x2_ref, code_ref, r_ref, o_ref):
  x = x2_ref[0][:, :D_MODEL]
  c8 = code_ref[0]
  ct = jnp.transpose(c8, (1, 0))
  c = ct[:, :1]
  xm = jnp.where(c != 0.0, x, 0.0)
  t = lax.dot_general(xm, r_ref[...], (((1,), (1,)), ((), ())),
                      preferred_element_type=jnp.float32)
  o_ref[0] = jnp.where(c == 2.0, t, xm)


def _tc_finish(x2, code8, r, n_rows):
  grid = n_rows // BTC
  return pl.pallas_call(
      _tc_finish_body,
      out_shape=jax.ShapeDtypeStruct((grid, BTC, D_MODEL), jnp.float32),
      grid=(grid,),
      in_specs=[
          pl.BlockSpec((1, BTC, 2 * D_MODEL), lambda i: (i, 0, 0)),
          pl.BlockSpec((1, 8, BTC), lambda i: (i, 0, 0)),
          pl.BlockSpec((D_MODEL, D_MODEL), lambda i: (0, 0)),
      ],
      out_specs=pl.BlockSpec((1, BTC, D_MODEL), lambda i: (i, 0, 0)),
  )(x2.reshape(grid, BTC, 2 * D_MODEL), code8, r)


def kernel(input_ids, role_mask, table, R):
  b, l = input_ids.shape
  n = b * l
  nw = 32
  ids_flat = input_ids.reshape(n).astype(jnp.int32)
  idx3d = ids_flat.reshape(nw, n // (nw * GRP), GRP)
  table2 = jnp.tile(table, (1, 2))
  x2 = _sc_gather_packed(table2, idx3d, n, nw)
  code = jnp.where(input_ids == PAD_IDX, 0.0,
                   1.0 + (role_mask == 1).astype(jnp.float32))
  code8 = jnp.broadcast_to(code.reshape(n // BTC, 1, BTC), (n // BTC, 8, BTC))
  out = _tc_finish(x2, code8, R, n)
  return out.reshape(b, l, D_MODEL)


# speedup vs baseline: 1.9646x; 1.9646x over previous
import functools

import jax
import jax.numpy as jnp
from jax import lax
from jax.experimental import pallas as pl
from jax.experimental.pallas import tpu as pltpu
from jax.experimental.pallas import tpu_sc as plsc

VOCAB = 1000000
D_MODEL = 64
PAD_IDX = 0
GRP = 128
GRP_PER_CHUNK = 5
CHUNK = GRP * GRP_PER_CHUNK
HALF = CHUNK // 2
K_SUB = 8
BLKV = 8192
VBLKS = -(-VOCAB // BLKV)


def _tc_format_table(table):
  table_t = table.T

  def body(t_ref, o_ref):
    xt = jnp.transpose(t_ref[...], (1, 0))
    o_ref[...] = jnp.concatenate(
        [xt[:BLKV // 2], xt[BLKV // 2:]], axis=1)

  return pl.pallas_call(
      body,
      out_shape=jax.ShapeDtypeStruct((VBLKS * BLKV // 2, 2 * D_MODEL),
                                     jnp.float32),
      grid=(VBLKS,),
      in_specs=[pl.BlockSpec((D_MODEL, BLKV), lambda b: (0, b))],
      out_specs=pl.BlockSpec((BLKV // 2, 2 * D_MODEL), lambda b: (b, 0)),
  )(table_t)


def _sc_gather_packed(tablep, idx3d, n_rows, nw):
  g_per_w = idx3d.shape[1]
  chunks = g_per_w // GRP_PER_CHUNK
  rows_chunk = GRP * GRP_PER_CHUNK
  n_tab = tablep.shape[0] * 2
  mesh = plsc.VectorSubcoreMesh(core_axis_name="c", subcore_axis_name="s")

  @functools.partial(
      pl.kernel,
      out_type=jax.ShapeDtypeStruct((n_rows // 2, 2 * D_MODEL), jnp.float32),
      mesh=mesh,
      scratch_types=[
          pltpu.VMEM((g_per_w, GRP), jnp.int32),
          pltpu.VMEM((rows_chunk, D_MODEL), jnp.float32),
          pltpu.VMEM((rows_chunk, D_MODEL), jnp.float32),
          pltpu.SemaphoreType.DMA,
          pltpu.SemaphoreType.DMA,
      ],
      compiler_params=pltpu.CompilerParams(use_tc_tiling_on_sc=False),
  )
  def k(table_hbm, idx_hbm, out_hbm, idx_v, rows_a, rows_b, sem_a, sem_b):
    wid = lax.axis_index("s") * plsc.get_sparse_core_info().num_cores + \
        lax.axis_index("c")
    pltpu.sync_copy(idx_hbm.at[wid], idx_v)
    row_base = wid * g_per_w * GRP // 2

    def gather_chunk(c, rows_v, sem):
      copies = []
      for j in range(GRP_PER_CHUNK):
        copies.append(pltpu.async_copy(
            table_hbm.at[idx_v.at[c * GRP_PER_CHUNK + j]],
            rows_v.at[pl.ds(j * GRP, GRP)], sem))
      return copies

    def drain_and_store(c, copies, rows_v):
      for cp in copies:
        cp.wait()
      p0 = row_base + c * HALF
      pltpu.sync_copy(rows_v.at[pl.ds(0, HALF)],
                      out_hbm.at[pl.ds(p0, HALF), pl.ds(0, D_MODEL)])
      pltpu.sync_copy(rows_v.at[pl.ds(HALF, HALF)],
                      out_hbm.at[pl.ds(p0, HALF), pl.ds(D_MODEL, D_MODEL)])

    copies = gather_chunk(0, rows_a, sem_a)
    for c in range(chunks):
      buf = rows_a if c % 2 == 0 else rows_b
      nbuf, nsem = (rows_b, sem_b) if c % 2 == 0 else (rows_a, sem_a)
      ncopies = gather_chunk(c + 1, nbuf, nsem) if c + 1 < chunks else []
      drain_and_store(c, copies, buf)
      copies = ncopies

  return k(tablep.reshape(n_tab, D_MODEL), idx3d)


def _tc_finish_body(x2_ref, code_ref, g_ref, o_ref):
  gmat = g_ref[...]
  for j in range(K_SUB):
    x2 = x2_ref[j]
    c8 = code_ref[j]
    ct = jnp.transpose(c8, (1, 0))
    ca = ct[:HALF, :1]
    cb = ct[HALF:, :1]
    lane = lax.broadcasted_iota(jnp.int32, (HALF, 2 * D_MODEL), 1)
    c2 = jnp.where(lane < D_MODEL, ca, cb)
    xm2 = jnp.where(c2 != 0.0, x2, 0.0)
    t2 = lax.dot_general(xm2, gmat, (((1,), (0,)), ((), ())),
                         preferred_element_type=jnp.float32)
    y2 = jnp.where(c2 == 2.0, t2, xm2)
    o_ref[j, :HALF] = y2[:, :D_MODEL]
    o_ref[j, HALF:] = y2[:, D_MODEL:]


def _tc_finish(x2, code8, g, n_rows):
  nc = n_rows // CHUNK
  grid = nc // K_SUB
  return pl.pallas_call(
      _tc_finish_body,
      out_shape=jax.ShapeDtypeStruct((nc, CHUNK, D_MODEL), jnp.float32),
      grid=(grid,),
      in_specs=[
          pl.BlockSpec((K_SUB, HALF, 2 * D_MODEL), lambda i: (i, 0, 0)),
          pl.BlockSpec((K_SUB, 8, CHUNK), lambda i: (i, 0, 0)),
          pl.BlockSpec((2 * D_MODEL, 2 * D_MODEL), lambda i: (0, 0)),
      ],
      out_specs=pl.BlockSpec((K_SUB, CHUNK, D_MODEL), lambda i: (i, 0, 0)),
  )(x2.reshape(nc, HALF, 2 * D_MODEL), code8, g)


def kernel(input_ids, role_mask, table, R):
  b, l = input_ids.shape
  n = b * l
  nc = n // CHUNK
  nw = 32
  ids = input_ids.reshape(n).astype(jnp.int32)
  vb = ids % BLKV
  g = 2 * ((BLKV // 2) * (ids // BLKV) + vb % (BLKV // 2)) \
      + vb // (BLKV // 2)
  idx3d = g.reshape(nw, n // (nw * GRP), GRP)
  tablep = _tc_format_table(table)
  x2 = _sc_gather_packed(tablep, idx3d, n, nw)
  code = jnp.where(input_ids == PAD_IDX, 0.0,
                   1.0 + (role_mask == 1).astype(jnp.float32))
  code8 = jnp.broadcast_to(code.reshape(nc, 1, CHUNK), (nc, 8, CHUNK))
  zero = jnp.zeros((D_MODEL, D_MODEL), jnp.float32)
  gm = jnp.block([[R.T, zero], [zero, R.T]])
  out = _tc_finish(x2, code8, gm, n)
  return out.reshape(b, l, D_MODEL)


# speedup vs baseline: 2.1862x; 1.1128x over previous
import functools

import jax
import jax.numpy as jnp
from jax import lax
from jax.experimental import pallas as pl
from jax.experimental.pallas import tpu as pltpu
from jax.experimental.pallas import tpu_sc as plsc

VOCAB = 1000000
D_MODEL = 64
PAD_IDX = 0
GRP = 128
GRP_PER_CHUNK = 5
CHUNK = GRP * GRP_PER_CHUNK
HALF = CHUNK // 2
K_SUB = 8
BLKV = 16384
VBLKS = -(-VOCAB // BLKV)


def _tc_format_table(table):
  table_t = table.T

  def body(t_ref, o_ref):
    xt = jnp.transpose(t_ref[...], (1, 0))
    o_ref[...] = jnp.concatenate(
        [xt[:BLKV // 2], xt[BLKV // 2:]], axis=1)

  return pl.pallas_call(
      body,
      out_shape=jax.ShapeDtypeStruct((VBLKS * BLKV // 2, 2 * D_MODEL),
                                     jnp.float32),
      grid=(VBLKS,),
      in_specs=[pl.BlockSpec((D_MODEL, BLKV), lambda b: (0, b))],
      out_specs=pl.BlockSpec((BLKV // 2, 2 * D_MODEL), lambda b: (b, 0)),
  )(table_t)


def _sc_gather_packed(tablep, idx3d, n_rows, nw):
  g_per_w = idx3d.shape[1]
  chunks = g_per_w // GRP_PER_CHUNK
  rows_chunk = GRP * GRP_PER_CHUNK
  n_tab = tablep.shape[0] * 2
  mesh = plsc.VectorSubcoreMesh(core_axis_name="c", subcore_axis_name="s")

  @functools.partial(
      pl.kernel,
      out_type=jax.ShapeDtypeStruct((n_rows // 2, 2 * D_MODEL), jnp.float32),
      mesh=mesh,
      scratch_types=[
          pltpu.VMEM((g_per_w, GRP), jnp.int32),
          pltpu.VMEM((rows_chunk, D_MODEL), jnp.float32),
          pltpu.VMEM((rows_chunk, D_MODEL), jnp.float32),
          pltpu.SemaphoreType.DMA,
          pltpu.SemaphoreType.DMA,
      ],
      compiler_params=pltpu.CompilerParams(use_tc_tiling_on_sc=False),
  )
  def k(table_hbm, idx_hbm, out_hbm, idx_v, rows_a, rows_b, sem_a, sem_b):
    wid = lax.axis_index("s") * plsc.get_sparse_core_info().num_cores + \
        lax.axis_index("c")
    pltpu.sync_copy(idx_hbm.at[wid], idx_v)
    row_base = wid * g_per_w * GRP // 2

    def gather_chunk(c, rows_v, sem):
      copies = []
      for j in range(GRP_PER_CHUNK):
        copies.append(pltpu.async_copy(
            table_hbm.at[idx_v.at[c * GRP_PER_CHUNK + j]],
            rows_v.at[pl.ds(j * GRP, GRP)], sem))
      return copies

    def drain_and_store(c, copies, rows_v):
      for cp in copies:
        cp.wait()
      p0 = row_base + c * HALF
      pltpu.sync_copy(rows_v.at[pl.ds(0, HALF)],
                      out_hbm.at[pl.ds(p0, HALF), pl.ds(0, D_MODEL)])
      pltpu.sync_copy(rows_v.at[pl.ds(HALF, HALF)],
                      out_hbm.at[pl.ds(p0, HALF), pl.ds(D_MODEL, D_MODEL)])

    copies = gather_chunk(0, rows_a, sem_a)
    for c in range(chunks):
      buf = rows_a if c % 2 == 0 else rows_b
      nbuf, nsem = (rows_b, sem_b) if c % 2 == 0 else (rows_a, sem_a)
      ncopies = gather_chunk(c + 1, nbuf, nsem) if c + 1 < chunks else []
      drain_and_store(c, copies, buf)
      copies = ncopies

  return k(tablep.reshape(n_tab, D_MODEL), idx3d)


def _tc_finish_body(x2_ref, code_ref, g_ref, o_ref):
  gmat = g_ref[...]
  ct = jnp.transpose(code_ref[...], (1, 0))
  lane = lax.broadcasted_iota(jnp.int32, (HALF, 2 * D_MODEL), 1)
  for j in range(K_SUB):
    x2 = x2_ref[j]
    ca = ct[:HALF, j:j + 1]
    cb = ct[HALF:, j:j + 1]
    c2 = jnp.where(lane < D_MODEL, ca, cb)
    xm2 = jnp.where(c2 != 0.0, x2, 0.0)
    t2 = lax.dot_general(xm2, gmat, (((1,), (0,)), ((), ())),
                         preferred_element_type=jnp.float32)
    y2 = jnp.where(c2 == 2.0, t2, xm2)
    o_ref[j, :HALF] = y2[:, :D_MODEL]
    o_ref[j, HALF:] = y2[:, D_MODEL:]


def _tc_finish(x2, code2d, g, n_rows):
  nc = n_rows // CHUNK
  grid = nc // K_SUB
  return pl.pallas_call(
      _tc_finish_body,
      out_shape=jax.ShapeDtypeStruct((nc, CHUNK, D_MODEL), jnp.float32),
      grid=(grid,),
      in_specs=[
          pl.BlockSpec((K_SUB, HALF, 2 * D_MODEL), lambda i: (i, 0, 0)),
          pl.BlockSpec((K_SUB, CHUNK), lambda i: (i, 0)),
          pl.BlockSpec((2 * D_MODEL, 2 * D_MODEL), lambda i: (0, 0)),
      ],
      out_specs=pl.BlockSpec((K_SUB, CHUNK, D_MODEL), lambda i: (i, 0, 0)),
  )(x2.reshape(nc, HALF, 2 * D_MODEL), code2d, g)


def kernel(input_ids, role_mask, table, R):
  b, l = input_ids.shape
  n = b * l
  nc = n // CHUNK
  nw = 32
  ids = input_ids.reshape(n).astype(jnp.int32)
  vb = ids % BLKV
  g = 2 * ((BLKV // 2) * (ids // BLKV) + vb % (BLKV // 2)) \
      + vb // (BLKV // 2)
  idx3d = g.reshape(nw, n // (nw * GRP), GRP)
  tablep = _tc_format_table(table)
  x2 = _sc_gather_packed(tablep, idx3d, n, nw)
  code = jnp.where(input_ids == PAD_IDX, 0.0,
                   1.0 + (role_mask == 1).astype(jnp.float32))
  code2d = code.reshape(nc, CHUNK)
  zero = jnp.zeros((D_MODEL, D_MODEL), jnp.float32)
  gm = jnp.block([[R.T, zero], [zero, R.T]])
  out = _tc_finish(x2, code2d, gm, n)
  return out.reshape(b, l, D_MODEL)


# speedup vs baseline: 2.2717x; 1.0391x over previous
import functools

import jax
import jax.numpy as jnp
from jax import lax
from jax.experimental import pallas as pl
from jax.experimental.pallas import tpu as pltpu
from jax.experimental.pallas import tpu_sc as plsc

VOCAB = 1000000
D_MODEL = 64
PAD_IDX = 0
GRP = 128
GRP_PER_CHUNK = 5
CHUNK = GRP * GRP_PER_CHUNK
HALF = CHUNK // 2
K_SUB = 8
BLKV = 32768
VBLKS = -(-VOCAB // BLKV)


def _tc_format_table(table):
  table_t = table.T

  def body(t_ref, o_ref):
    xt = jnp.transpose(t_ref[...], (1, 0))
    o_ref[...] = jnp.concatenate(
        [xt[:BLKV // 2], xt[BLKV // 2:]], axis=1)

  return pl.pallas_call(
      body,
      out_shape=jax.ShapeDtypeStruct((VBLKS * BLKV // 2, 2 * D_MODEL),
                                     jnp.float32),
      grid=(VBLKS,),
      in_specs=[pl.BlockSpec((D_MODEL, BLKV), lambda b: (0, b))],
      out_specs=pl.BlockSpec((BLKV // 2, 2 * D_MODEL), lambda b: (b, 0)),
  )(table_t)


def _sc_gather_packed(tablep, idx3d, n_rows, nw):
  g_per_w = idx3d.shape[1]
  chunks = g_per_w // GRP_PER_CHUNK
  rows_chunk = GRP * GRP_PER_CHUNK
  n_tab = tablep.shape[0] * 2
  mesh = plsc.VectorSubcoreMesh(core_axis_name="c", subcore_axis_name="s")

  @functools.partial(
      pl.kernel,
      out_type=jax.ShapeDtypeStruct((n_rows // 2, 2 * D_MODEL), jnp.float32),
      mesh=mesh,
      scratch_types=[
          pltpu.VMEM((g_per_w, GRP), jnp.int32),
          pltpu.VMEM((rows_chunk, D_MODEL), jnp.float32),
          pltpu.VMEM((rows_chunk, D_MODEL), jnp.float32),
          pltpu.SemaphoreType.DMA,
          pltpu.SemaphoreType.DMA,
      ],
      compiler_params=pltpu.CompilerParams(use_tc_tiling_on_sc=False),
  )
  def k(table_hbm, idx_hbm, out_hbm, idx_v, rows_a, rows_b, sem_a, sem_b):
    wid = lax.axis_index("s") * plsc.get_sparse_core_info().num_cores + \
        lax.axis_index("c")
    pltpu.sync_copy(idx_hbm.at[wid], idx_v)
    row_base = wid * g_per_w * GRP // 2

    def gather_chunk(c, rows_v, sem):
      copies = []
      for j in range(GRP_PER_CHUNK):
        copies.append(pltpu.async_copy(
            table_hbm.at[idx_v.at[c * GRP_PER_CHUNK + j]],
            rows_v.at[pl.ds(j * GRP, GRP)], sem))
      return copies

    def drain_and_store(c, copies, rows_v):
      for cp in copies:
        cp.wait()
      p0 = row_base + c * HALF
      pltpu.sync_copy(rows_v.at[pl.ds(0, HALF)],
                      out_hbm.at[pl.ds(p0, HALF), pl.ds(0, D_MODEL)])
      pltpu.sync_copy(rows_v.at[pl.ds(HALF, HALF)],
                      out_hbm.at[pl.ds(p0, HALF), pl.ds(D_MODEL, D_MODEL)])

    copies = gather_chunk(0, rows_a, sem_a)
    for c in range(chunks):
      buf = rows_a if c % 2 == 0 else rows_b
      nbuf, nsem = (rows_b, sem_b) if c % 2 == 0 else (rows_a, sem_a)
      ncopies = gather_chunk(c + 1, nbuf, nsem) if c + 1 < chunks else []
      drain_and_store(c, copies, buf)
      copies = ncopies

  return k(tablep.reshape(n_tab, D_MODEL), idx3d)


def _tc_finish_body(x2_ref, code_ref, g_ref, o_ref):
  gmat = g_ref[...]
  ct = jnp.transpose(code_ref[...], (1, 0))
  lane = lax.broadcasted_iota(jnp.int32, (HALF, 2 * D_MODEL), 1)
  for j in range(K_SUB):
    x2 = x2_ref[j]
    col = jnp.concatenate(
        [ct[:, 5 * j + q:5 * j + q + 1] for q in range(5)], axis=0)
    ca = col[:HALF]
    cb = col[HALF:]
    c2 = jnp.where(lane < D_MODEL, ca, cb)
    xm2 = jnp.where(c2 != 0.0, x2, 0.0)
    t2 = lax.dot_general(xm2, gmat, (((1,), (0,)), ((), ())),
                         preferred_element_type=jnp.float32)
    y2 = jnp.where(c2 == 2.0, t2, xm2)
    o_ref[j, :HALF] = y2[:, :D_MODEL]
    o_ref[j, HALF:] = y2[:, D_MODEL:]


def _tc_finish(x2, code2d, g, n_rows):
  nc = n_rows // CHUNK
  grid = nc // K_SUB
  return pl.pallas_call(
      _tc_finish_body,
      out_shape=jax.ShapeDtypeStruct((nc, CHUNK, D_MODEL), jnp.float32),
      grid=(grid,),
      in_specs=[
          pl.BlockSpec((K_SUB, HALF, 2 * D_MODEL), lambda i: (i, 0, 0)),
          pl.BlockSpec((K_SUB * CHUNK // GRP, GRP), lambda i: (i, 0)),
          pl.BlockSpec((2 * D_MODEL, 2 * D_MODEL), lambda i: (0, 0)),
      ],
      out_specs=pl.BlockSpec((K_SUB, CHUNK, D_MODEL), lambda i: (i, 0, 0)),
  )(x2.reshape(nc, HALF, 2 * D_MODEL), code2d, g)


def kernel(input_ids, role_mask, table, R):
  b, l = input_ids.shape
  n = b * l
  nc = n // CHUNK
  nw = 32
  ids = input_ids.reshape(n).astype(jnp.int32)
  vb = ids % BLKV
  g = 2 * ((BLKV // 2) * (ids // BLKV) + vb % (BLKV // 2)) \
      + vb // (BLKV // 2)
  idx3d = g.reshape(nw, n // (nw * GRP), GRP)
  tablep = _tc_format_table(table)
  x2 = _sc_gather_packed(tablep, idx3d, n, nw)
  code = jnp.where(input_ids == PAD_IDX, 0.0,
                   1.0 + (role_mask == 1).astype(jnp.float32))
  code2d = code.reshape(n // GRP, GRP)
  zero = jnp.zeros((D_MODEL, D_MODEL), jnp.float32)
  gm = jnp.block([[R.T, zero], [zero, R.T]])
  out = _tc_finish(x2, code2d, gm, n)
  return out.reshape(b, l, D_MODEL)


# speedup vs baseline: 2.2782x; 1.0028x over previous
import functools

import jax
import jax.numpy as jnp
from jax import lax
from jax.experimental import pallas as pl
from jax.experimental.pallas import tpu as pltpu
from jax.experimental.pallas import tpu_sc as plsc

VOCAB = 1000000
D_MODEL = 64
PAD_IDX = 0
GRP = 128
GRP_PER_CHUNK = 5
CHUNK = GRP * GRP_PER_CHUNK
HALF = CHUNK // 2
K_SUB = 16
BLKV = 32768
VBLKS = -(-VOCAB // BLKV)


def _tc_format_table(table):
  table_t = table.T

  def body(t_ref, o_ref):
    xt = jnp.transpose(t_ref[...], (1, 0))
    o_ref[...] = jnp.concatenate(
        [xt[:BLKV // 2], xt[BLKV // 2:]], axis=1)

  return pl.pallas_call(
      body,
      out_shape=jax.ShapeDtypeStruct((VBLKS * BLKV // 2, 2 * D_MODEL),
                                     jnp.float32),
      grid=(VBLKS,),
      in_specs=[pl.BlockSpec((D_MODEL, BLKV), lambda b: (0, b))],
      out_specs=pl.BlockSpec((BLKV // 2, 2 * D_MODEL), lambda b: (b, 0)),
  )(table_t)


def _sc_gather_packed(tablep, idx3d, n_rows, nw):
  g_per_w = idx3d.shape[1]
  chunks = g_per_w // GRP_PER_CHUNK
  rows_chunk = GRP * GRP_PER_CHUNK
  n_tab = tablep.shape[0] * 2
  mesh = plsc.VectorSubcoreMesh(core_axis_name="c", subcore_axis_name="s")

  @functools.partial(
      pl.kernel,
      out_type=jax.ShapeDtypeStruct((n_rows // 2, 2 * D_MODEL), jnp.float32),
      mesh=mesh,
      scratch_types=[
          pltpu.VMEM((g_per_w, GRP), jnp.int32),
          pltpu.VMEM((rows_chunk, D_MODEL), jnp.float32),
          pltpu.VMEM((rows_chunk, D_MODEL), jnp.float32),
          pltpu.SemaphoreType.DMA,
          pltpu.SemaphoreType.DMA,
      ],
      compiler_params=pltpu.CompilerParams(use_tc_tiling_on_sc=False),
  )
  def k(table_hbm, idx_hbm, out_hbm, idx_v, rows_a, rows_b, sem_a, sem_b):
    wid = lax.axis_index("s") * plsc.get_sparse_core_info().num_cores + \
        lax.axis_index("c")
    pltpu.sync_copy(idx_hbm.at[wid], idx_v)
    row_base = wid * g_per_w * GRP // 2

    def gather_chunk(c, rows_v, sem):
      copies = []
      for j in range(GRP_PER_CHUNK):
        copies.append(pltpu.async_copy(
            table_hbm.at[idx_v.at[c * GRP_PER_CHUNK + j]],
            rows_v.at[pl.ds(j * GRP, GRP)], sem))
      return copies

    def drain_and_store(c, copies, rows_v):
      for cp in copies:
        cp.wait()
      p0 = row_base + c * HALF
      pltpu.sync_copy(rows_v.at[pl.ds(0, HALF)],
                      out_hbm.at[pl.ds(p0, HALF), pl.ds(0, D_MODEL)])
      pltpu.sync_copy(rows_v.at[pl.ds(HALF, HALF)],
                      out_hbm.at[pl.ds(p0, HALF), pl.ds(D_MODEL, D_MODEL)])

    copies = gather_chunk(0, rows_a, sem_a)
    for c in range(chunks):
      buf = rows_a if c % 2 == 0 else rows_b
      nbuf, nsem = (rows_b, sem_b) if c % 2 == 0 else (rows_a, sem_a)
      ncopies = gather_chunk(c + 1, nbuf, nsem) if c + 1 < chunks else []
      drain_and_store(c, copies, buf)
      copies = ncopies

  return k(tablep.reshape(n_tab, D_MODEL), idx3d)


def _tc_finish_body(x2_ref, code_ref, g_ref, o_ref):
  gmat = g_ref[...]
  ct = jnp.transpose(code_ref[...], (1, 0))
  lane = lax.broadcasted_iota(jnp.int32, (HALF, 2 * D_MODEL), 1)
  for j in range(K_SUB):
    x2 = x2_ref[j]
    col = jnp.concatenate(
        [ct[:, 5 * j + q:5 * j + q + 1] for q in range(5)], axis=0)
    ca = col[:HALF]
    cb = col[HALF:]
    c2 = jnp.where(lane < D_MODEL, ca, cb)
    xm2 = jnp.where(c2 != 0.0, x2, 0.0)
    t2 = lax.dot_general(xm2, gmat, (((1,), (0,)), ((), ())),
                         preferred_element_type=jnp.float32)
    y2 = jnp.where(c2 == 2.0, t2, xm2)
    o_ref[j, :HALF] = y2[:, :D_MODEL]
    o_ref[j, HALF:] = y2[:, D_MODEL:]


def _tc_finish(x2, code2d, g, n_rows):
  nc = n_rows // CHUNK
  grid = nc // K_SUB
  return pl.pallas_call(
      _tc_finish_body,
      out_shape=jax.ShapeDtypeStruct((nc, CHUNK, D_MODEL), jnp.float32),
      grid=(grid,),
      in_specs=[
          pl.BlockSpec((K_SUB, HALF, 2 * D_MODEL), lambda i: (i, 0, 0)),
          pl.BlockSpec((K_SUB * CHUNK // GRP, GRP), lambda i: (i, 0)),
          pl.BlockSpec((2 * D_MODEL, 2 * D_MODEL), lambda i: (0, 0)),
      ],
      out_specs=pl.BlockSpec((K_SUB, CHUNK, D_MODEL), lambda i: (i, 0, 0)),
  )(x2.reshape(nc, HALF, 2 * D_MODEL), code2d, g)


def kernel(input_ids, role_mask, table, R):
  b, l = input_ids.shape
  n = b * l
  nc = n // CHUNK
  nw = 32
  ids = input_ids.reshape(n).astype(jnp.int32)
  vb = ids % BLKV
  g = 2 * ((BLKV // 2) * (ids // BLKV) + vb % (BLKV // 2)) \
      + vb // (BLKV // 2)
  idx3d = g.reshape(nw, n // (nw * GRP), GRP)
  tablep = _tc_format_table(table)
  x2 = _sc_gather_packed(tablep, idx3d, n, nw)
  code = jnp.where(input_ids == PAD_IDX, 0.0,
                   1.0 + (role_mask == 1).astype(jnp.float32))
  code2d = code.reshape(n // GRP, GRP)
  zero = jnp.zeros((D_MODEL, D_MODEL), jnp.float32)
  gm = jnp.block([[R.T, zero], [zero, R.T]])
  out = _tc_finish(x2, code2d, gm, n)
  return out.reshape(b, l, D_MODEL)
